# Initial kernel scaffold; baseline (speedup 1.0000x reference)
import jax
import jax.numpy as jnp
from jax.experimental import pallas as pl


def kernel(data, starts, length):
    raise NotImplementedError("write your pallas kernel here")


# speedup vs baseline: 477.0983x; 477.0983x over previous
import functools

import jax
import jax.numpy as jnp
from jax import lax
from jax.experimental import pallas as pl
from jax.experimental.pallas import tpu as pltpu
from jax.experimental.pallas import tpu_sc as plsc

N = 108000
B = 1024
L = 4096

_info = plsc.get_sparse_core_info()
_NC = _info.num_cores
_NS = _info.num_subcores
_NW = _NC * _NS
_ROWS = B // _NW
_VECS = L // 16


def _make_kernel():
    mesh = plsc.VectorSubcoreMesh(core_axis_name="c", subcore_axis_name="s")

    @functools.partial(
        pl.kernel,
        out_type=jax.ShapeDtypeStruct((B, L), jnp.float32),
        mesh=mesh,
        scratch_types=[
            pltpu.VMEM((N,), jnp.float32),
            pltpu.VMEM((_ROWS + 16,), jnp.int32),
            pltpu.VMEM((L,), jnp.float32),
            pltpu.SemaphoreType.DMA,
        ],
    )
    def gather_windows(data_hbm, starts_hbm, out_hbm, data_v, starts_v, row_v, sem):
        wid = lax.axis_index("s") * _NC + lax.axis_index("c")
        base = wid * _ROWS
        pltpu.sync_copy(starts_hbm.at[pl.ds(base, _ROWS)],
                        starts_v.at[pl.ds(0, _ROWS)])
        pltpu.sync_copy(data_hbm, data_v)

        def row_body(i, carry):
            s = starts_v[pl.ds(i, 16)][0]

            def vec_body(k, carry2):
                row_v[pl.ds(k * 16, 16)] = data_v[pl.ds(s + k * 16, 16)]
                return carry2

            lax.fori_loop(0, _VECS, vec_body, 0, unroll=8)
            pltpu.sync_copy(row_v, out_hbm.at[base + i])
            return carry

        lax.fori_loop(0, _ROWS, row_body, 0)

    return gather_windows


_gather = _make_kernel()


@jax.jit
def _run(data, starts):
    out = _gather(data.reshape(N), starts.astype(jnp.int32))
    return out.reshape(B, L, 1)


def kernel(data, starts, length):
    del length
    return _run(data, starts)


# speedup vs baseline: 485.8042x; 1.0182x over previous
import functools

import jax
import jax.numpy as jnp
from jax import lax
from jax.experimental import pallas as pl
from jax.experimental.pallas import tpu as pltpu
from jax.experimental.pallas import tpu_sc as plsc

N = 108000
B = 1024
L = 4096

_info = plsc.get_sparse_core_info()
_NC = _info.num_cores
_NS = _info.num_subcores
_NW = _NC * _NS
_ROWS = B // _NW
_VECS = L // 16
_NBUF = 4
_GROUPS = _ROWS // _NBUF


def _make_kernel():
    mesh = plsc.VectorSubcoreMesh(core_axis_name="c", subcore_axis_name="s")

    @functools.partial(
        pl.kernel,
        out_type=jax.ShapeDtypeStruct((B, L), jnp.float32),
        mesh=mesh,
        scratch_types=[
            pltpu.VMEM((N,), jnp.float32),
            pltpu.VMEM((_ROWS + 16,), jnp.int32),
            pltpu.VMEM((_NBUF, L), jnp.float32),
            pltpu.SemaphoreType.DMA,
        ],
    )
    def gather_windows(data_hbm, starts_hbm, out_hbm, data_v, starts_v, rows_v,
                       sem):
        wid = lax.axis_index("s") * _NC + lax.axis_index("c")
        base = wid * _ROWS
        pltpu.sync_copy(starts_hbm.at[pl.ds(base, _ROWS)],
                        starts_v.at[pl.ds(0, _ROWS)])
        pltpu.sync_copy(data_hbm, data_v)

        def fill(i, b):
            s = starts_v[pl.ds(i, 16)][0]

            def vec_body(k, carry):
                rows_v[b, pl.ds(k * 16, 16)] = data_v[pl.ds(s + k * 16, 16)]
                return carry

            lax.fori_loop(0, _VECS, vec_body, 0, unroll=8)

        for b in range(_NBUF):
            fill(b, b)
            pltpu.async_copy(rows_v.at[b], out_hbm.at[base + b], sem)

        def group_body(g, carry):
            for b in range(_NBUF):
                i = g * _NBUF + b
                pltpu.make_async_copy(data_hbm.at[pl.ds(0, L)], rows_v.at[b],
                                      sem).wait()
                fill(i, b)
                pltpu.async_copy(rows_v.at[b], out_hbm.at[base + i], sem)
            return carry

        lax.fori_loop(1, _GROUPS, group_body, 0)

        for b in range(_NBUF):
            pltpu.make_async_copy(data_hbm.at[pl.ds(0, L)], rows_v.at[b],
                                  sem).wait()

    return gather_windows


_gather = _make_kernel()


@jax.jit
def _run(data, starts):
    out = _gather(data.reshape(N), starts.astype(jnp.int32))
    return out.reshape(B, L, 1)


def kernel(data, starts, length):
    del length
    return _run(data, starts)


# speedup vs baseline: 683.8286x; 1.4076x over previous
import functools

import jax
import jax.numpy as jnp
from jax import lax
from jax.experimental import pallas as pl
from jax.experimental.pallas import tpu as pltpu
from jax.experimental.pallas import tpu_sc as plsc

N = 108000
B = 1024
L = 4096

_info = plsc.get_sparse_core_info()
_NC = _info.num_cores
_NS = _info.num_subcores
_NW = _NC * _NS
_ROWS = B // _NW
_VECS = L // 16
_NBUF = 4
_GROUPS = _ROWS // _NBUF


def _make_kernel():
    mesh = plsc.VectorSubcoreMesh(core_axis_name="c", subcore_axis_name="s")

    @functools.partial(
        pl.kernel,
        out_type=jax.ShapeDtypeStruct((B, L), jnp.float32),
        mesh=mesh,
        scratch_types=[
            pltpu.VMEM((N,), jnp.float32),
            pltpu.VMEM((_ROWS + 16,), jnp.int32),
            pltpu.VMEM((_NBUF, L), jnp.float32),
            pltpu.SemaphoreType.DMA,
        ],
    )
    def gather_windows(data_hbm, starts_hbm, out_hbm, data_v, starts_v, rows_v,
                       sem):
        wid = lax.axis_index("s") * _NC + lax.axis_index("c")
        base = wid * _ROWS
        pltpu.sync_copy(starts_hbm.at[pl.ds(base, _ROWS)],
                        starts_v.at[pl.ds(0, _ROWS)])
        pltpu.sync_copy(data_hbm, data_v)

        def fill(i, b):
            s = starts_v[pl.ds(i, 16)][0]

            @plsc.parallel_loop(0, L, step=16, unroll=8)
            def vec_body(o):
                rows_v[b, pl.ds(o, 16)] = data_v[pl.ds(s + o, 16)]

        for b in range(_NBUF):
            fill(b, b)
            pltpu.async_copy(rows_v.at[b], out_hbm.at[base + b], sem)

        def group_body(g, carry):
            for b in range(_NBUF):
                i = g * _NBUF + b
                pltpu.make_async_copy(data_hbm.at[pl.ds(0, L)], rows_v.at[b],
                                      sem).wait()
                fill(i, b)
                pltpu.async_copy(rows_v.at[b], out_hbm.at[base + i], sem)
            return carry

        lax.fori_loop(1, _GROUPS, group_body, 0)

        for b in range(_NBUF):
            pltpu.make_async_copy(data_hbm.at[pl.ds(0, L)], rows_v.at[b],
                                  sem).wait()

    return gather_windows


_gather = _make_kernel()


@jax.jit
def _run(data, starts):
    out = _gather(data.reshape(N), starts.astype(jnp.int32))
    return out.reshape(B, L, 1)


def kernel(data, starts, length):
    del length
    return _run(data, starts)


# speedup vs baseline: 685.4982x; 1.0024x over previous
import functools

import jax
import jax.numpy as jnp
from jax import lax
from jax.experimental import pallas as pl
from jax.experimental.pallas import tpu as pltpu
from jax.experimental.pallas import tpu_sc as plsc

N = 108000
B = 1024
L = 4096

_info = plsc.get_sparse_core_info()
_NC = _info.num_cores
_NS = _info.num_subcores
_NW = _NC * _NS
_ROWS = B // _NW
_VECS = L // 16
_NBUF = 4
_GROUPS = _ROWS // _NBUF


def _make_kernel():
    mesh = plsc.VectorSubcoreMesh(core_axis_name="c", subcore_axis_name="s")

    @functools.partial(
        pl.kernel,
        out_type=jax.ShapeDtypeStruct((B, L), jnp.float32),
        mesh=mesh,
        scratch_types=[
            pltpu.VMEM((N,), jnp.float32),
            pltpu.VMEM((_ROWS + 16,), jnp.int32),
            pltpu.VMEM((_NBUF, L), jnp.float32),
            pltpu.SemaphoreType.DMA,
        ],
    )
    def gather_windows(data_hbm, starts_hbm, out_hbm, data_v, starts_v, rows_v,
                       sem):
        wid = lax.axis_index("s") * _NC + lax.axis_index("c")
        base = wid * _ROWS
        pltpu.sync_copy(starts_hbm.at[pl.ds(base, _ROWS)],
                        starts_v.at[pl.ds(0, _ROWS)])
        pltpu.sync_copy(data_hbm, data_v)

        def fill(i, b):
            s = starts_v[pl.ds(i, 16)][0]

            @plsc.parallel_loop(0, L, step=16, unroll=16)
            def vec_body(o):
                rows_v[b, pl.ds(o, 16)] = data_v[pl.ds(s + o, 16)]

        for b in range(_NBUF):
            fill(b, b)
            pltpu.async_copy(rows_v.at[b], out_hbm.at[base + b], sem)

        def group_body(g, carry):
            for b in range(_NBUF):
                i = g * _NBUF + b
                pltpu.make_async_copy(data_hbm.at[pl.ds(0, L)], rows_v.at[b],
                                      sem).wait()
                fill(i, b)
                pltpu.async_copy(rows_v.at[b], out_hbm.at[base + i], sem)
            return carry

        lax.fori_loop(1, _GROUPS, group_body, 0)

        for b in range(_NBUF):
            pltpu.make_async_copy(data_hbm.at[pl.ds(0, L)], rows_v.at[b],
                                  sem).wait()

    return gather_windows


_gather = _make_kernel()


@jax.jit
def _run(data, starts):
    out = _gather(data.reshape(N), starts.astype(jnp.int32))
    return out.reshape(B, L, 1)


def kernel(data, starts, length):
    del length
    return _run(data, starts)


# speedup vs baseline: 971.2503x; 1.4169x over previous
import functools

import jax
import jax.numpy as jnp
from jax import lax
from jax.experimental import pallas as pl
from jax.experimental.pallas import tpu as pltpu
from jax.experimental.pallas import tpu_sc as plsc

N = 108000
B = 1024
L = 4096

_info = plsc.get_sparse_core_info()
_NC = _info.num_cores
_NS = _info.num_subcores
_NW = _NC * _NS
_ROWS = B // _NW
_VECS = L // 16
_NBUF = 4
_GROUPS = _ROWS // _NBUF


def _make_kernel():
    mesh = plsc.VectorSubcoreMesh(core_axis_name="c", subcore_axis_name="s")

    @functools.partial(
        pl.kernel,
        out_type=jax.ShapeDtypeStruct((B * L,), jnp.float32),
        mesh=mesh,
        scratch_types=[
            pltpu.VMEM((N,), jnp.float32),
            pltpu.VMEM((_ROWS + 16,), jnp.int32),
            pltpu.VMEM((_NBUF, L), jnp.float32),
            pltpu.SemaphoreType.DMA,
        ],
    )
    def gather_windows(data_hbm, starts_hbm, out_hbm, data_v, starts_v, rows_v,
                       sem):
        wid = lax.axis_index("s") * _NC + lax.axis_index("c")
        base = wid * _ROWS
        pltpu.sync_copy(starts_hbm.at[pl.ds(base, _ROWS)],
                        starts_v.at[pl.ds(0, _ROWS)])
        pltpu.sync_copy(data_hbm, data_v)

        def fill(i, b):
            s = starts_v[pl.ds(i, 16)][0]

            @plsc.parallel_loop(0, L, step=16, unroll=16)
            def vec_body(o):
                rows_v[b, pl.ds(o, 16)] = data_v[pl.ds(s + o, 16)]

        for b in range(_NBUF):
            fill(b, b)
            pltpu.async_copy(rows_v.at[b], out_hbm.at[pl.ds((base + b) * L, L)],
                             sem)

        def group_body(g, carry):
            for b in range(_NBUF):
                i = g * _NBUF + b
                pltpu.make_async_copy(data_hbm.at[pl.ds(0, L)], rows_v.at[b],
                                      sem).wait()
                fill(i, b)
                pltpu.async_copy(rows_v.at[b],
                                 out_hbm.at[pl.ds((base + i) * L, L)], sem)
            return carry

        lax.fori_loop(1, _GROUPS, group_body, 0)

        for b in range(_NBUF):
            pltpu.make_async_copy(data_hbm.at[pl.ds(0, L)], rows_v.at[b],
                                  sem).wait()

    return gather_windows


_gather = _make_kernel()


@jax.jit
def _run(data, starts):
    out = _gather(data.reshape(N), starts.astype(jnp.int32))
    return out.reshape(B, L, 1)


def kernel(data, starts, length):
    del length
    return _run(data, starts)
